# Initial kernel scaffold; baseline (speedup 1.0000x reference)
import jax
import jax.numpy as jnp
from jax.experimental import pallas as pl


def kernel(token_ids, table, pe):
    raise NotImplementedError("write your pallas kernel here")


# speedup vs baseline: 2.4608x; 2.4608x over previous
import functools
import jax
import jax.numpy as jnp
from jax import lax
from jax.experimental import pallas as pl
from jax.experimental.pallas import tpu as pltpu
from jax.experimental.pallas import tpu_sc as plsc

_VOCAB = 100000
_D = 64
_S = 200
_B = 4096
_NC = 2
_NS = 16
_NW = _NC * _NS
_ROWS = _B * _S
_SEQ_PER_W = _B // _NW
_NCH = 4
_CHUNK_ROWS = _NCH * _S
_IDXW = 100
_GATHERS = _CHUNK_ROWS // _IDXW
_N_CHUNKS = _SEQ_PER_W // _NCH
_SCALE = 8.0
_VECS_PER_ROW = _D // 16


def _sc_body(idx_hbm, pe_hbm, table_hbm, out_hbm, idx_v, rows_v, pe_v, sem):
    cid = lax.axis_index("c")
    sid = lax.axis_index("s")
    wid = sid * _NC + cid
    base_row = wid * (_SEQ_PER_W * _S)

    pltpu.sync_copy(pe_hbm, pe_v)

    def chunk_body(c, carry):
        row0 = base_row + c * _CHUNK_ROWS
        idx_off = pl.multiple_of(row0 // _IDXW, _GATHERS)
        pltpu.sync_copy(idx_hbm.at[pl.ds(idx_off, _GATHERS)], idx_v)
        for j in range(_GATHERS):
            pltpu.async_copy(
                table_hbm.at[idx_v.at[j]],
                rows_v.at[pl.ds(j * _IDXW, _IDXW)],
                sem,
            )
        for j in range(_GATHERS):
            pltpu.make_async_copy(
                table_hbm.at[idx_v.at[j]],
                rows_v.at[pl.ds(j * _IDXW, _IDXW)],
                sem,
            ).wait()

        def row_body(i, carry2):
            pr = lax.rem(i, _S)
            for k in range(_VECS_PER_ROW):
                sl = pl.ds(k * 16, 16)
                rows_v[i, sl] = rows_v[i, sl] * _SCALE + pe_v[pr, sl]
            return carry2

        lax.fori_loop(0, _CHUNK_ROWS, row_body, 0, unroll=2)

        pltpu.sync_copy(rows_v, out_hbm.at[pl.ds(row0, _CHUNK_ROWS)])
        return carry

    lax.fori_loop(0, _N_CHUNKS, chunk_body, 0)


@jax.jit
def _music_embedding(idx2d, pe_s, table):
    mesh = plsc.VectorSubcoreMesh(
        core_axis_name="c", subcore_axis_name="s",
        num_cores=_NC, num_subcores=_NS,
    )
    run = pl.kernel(
        _sc_body,
        out_type=jax.ShapeDtypeStruct((_ROWS, _D), jnp.float32),
        mesh=mesh,
        scratch_types=[
            pltpu.VMEM((_GATHERS, _IDXW), jnp.int32),
            pltpu.VMEM((_CHUNK_ROWS, _D), jnp.float32),
            pltpu.VMEM((_S, _D), jnp.float32),
            pltpu.SemaphoreType.DMA,
        ],
        compiler_params=pltpu.CompilerParams(use_tc_tiling_on_sc=False),
    )
    return run(idx2d, pe_s, table)


def kernel(token_ids, table, pe):
    idx2d = token_ids.reshape(_ROWS // _IDXW, _IDXW).astype(jnp.int32)
    pe_s = pe[:_S]
    out = _music_embedding(idx2d, pe_s, table)
    return out.reshape(_B, _S, _D)


# speedup vs baseline: 2.8449x; 1.1561x over previous
import jax
import jax.numpy as jnp
from jax import lax
from jax.experimental import pallas as pl
from jax.experimental.pallas import tpu as pltpu
from jax.experimental.pallas import tpu_sc as plsc

_VOCAB = 100000
_D = 64
_S = 200
_B = 4096
_NC = 2
_NS = 16
_NW = _NC * _NS
_ROWS = _B * _S
_SEQ_PER_W = _B // _NW
_IDXW = 100
_GATHERS = _S // _IDXW
_NBUF = 4
_PDIST = _NBUF - 1
_NGROUPS = _SEQ_PER_W // _NBUF
_SCALE = 8.0
_VECS_PER_ROW = _D // 16


def _sc_body(idx_hbm, pe_hbm, table_hbm, out_hbm,
             idx_v, pe_v, r0, r1, r2, r3,
             sg0, sg1, sg2, sg3, sw0, sw1, sw2, sw3):
    rows = (r0, r1, r2, r3)
    sem_g = (sg0, sg1, sg2, sg3)
    sem_w = (sw0, sw1, sw2, sw3)
    cid = lax.axis_index("c")
    sid = lax.axis_index("s")
    wid = sid * _NC + cid
    base_seq = wid * _SEQ_PER_W

    pltpu.sync_copy(pe_hbm, pe_v)
    pltpu.sync_copy(idx_hbm.at[pl.ds(base_seq, _SEQ_PER_W)], idx_v)

    def fire_gather(c, b):
        for j in range(_GATHERS):
            pltpu.async_copy(
                table_hbm.at[idx_v.at[c, j]],
                rows[b].at[pl.ds(j * _IDXW, _IDXW)],
                sem_g[b],
            )

    def drain_gather(c, b):
        for j in range(_GATHERS):
            pltpu.make_async_copy(
                table_hbm.at[idx_v.at[c, j]],
                rows[b].at[pl.ds(j * _IDXW, _IDXW)],
                sem_g[b],
            ).wait()

    def out_copy(c, b):
        return pltpu.make_async_copy(
            rows[b], out_hbm.at[pl.ds((base_seq + c) * _S, _S)], sem_w[b])

    for b in range(_PDIST):
        fire_gather(b, b)

    def group_body(g, carry):
        c0 = g * _NBUF
        for b in range(_NBUF):
            c = c0 + b
            drain_gather(c, b)

            def row_body(i, carry2):
                for k in range(_VECS_PER_ROW):
                    sl = pl.ds(k * 16, 16)
                    rows[b][i, sl] = rows[b][i, sl] * _SCALE + pe_v[i, sl]
                return carry2

            lax.fori_loop(0, _S, row_body, 0, unroll=2)
            out_copy(c, b).start()

            f = c + _PDIST
            fb = (b + _PDIST) % _NBUF
            if b == 0:
                @pl.when(g >= 1)
                def _():
                    out_copy(f - _NBUF, fb).wait()
                fire_gather(f, fb)
            else:
                @pl.when(g < _NGROUPS - 1)
                def _():
                    out_copy(f - _NBUF, fb).wait()
                    fire_gather(f, fb)
        return carry

    lax.fori_loop(0, _NGROUPS, group_body, 0)

    for b in range(_NBUF):
        out_copy(_SEQ_PER_W - _NBUF + b, b).wait()


@jax.jit
def _music_embedding(idx3d, pe_s, table):
    mesh = plsc.VectorSubcoreMesh(
        core_axis_name="c", subcore_axis_name="s",
        num_cores=_NC, num_subcores=_NS,
    )
    run = pl.kernel(
        _sc_body,
        out_type=jax.ShapeDtypeStruct((_ROWS, _D), jnp.float32),
        mesh=mesh,
        scratch_types=(
            [pltpu.VMEM((_SEQ_PER_W, _GATHERS, _IDXW), jnp.int32),
             pltpu.VMEM((_S, _D), jnp.float32)]
            + [pltpu.VMEM((_S, _D), jnp.float32) for _ in range(_NBUF)]
            + [pltpu.SemaphoreType.DMA for _ in range(2 * _NBUF)]
        ),
        compiler_params=pltpu.CompilerParams(use_tc_tiling_on_sc=False),
    )
    return run(idx3d, pe_s, table)


def kernel(token_ids, table, pe):
    idx3d = token_ids.reshape(_B, _GATHERS, _IDXW).astype(jnp.int32)
    pe_s = pe[:_S]
    out = _music_embedding(idx3d, pe_s, table)
    return out.reshape(_B, _S, _D)


# speedup vs baseline: 4.1431x; 1.4563x over previous
import jax
import jax.numpy as jnp
from jax import lax
from jax.experimental import pallas as pl
from jax.experimental.pallas import tpu as pltpu
from jax.experimental.pallas import tpu_sc as plsc

_VOCAB = 100000
_D = 64
_S = 200
_B = 4096
_NC = 2
_NS = 16
_NW = _NC * _NS
_ROWS = _B * _S
_SEQ_PER_W = _B // _NW
_IDXW = 100
_GATHERS = _S // _IDXW
_NBUF = 4
_PDIST = _NBUF - 1
_NGROUPS = _SEQ_PER_W // _NBUF
_SCALE = 8.0
_VECS_PER_ROW = _D // 16


def _sc_body(idx_hbm, pe_hbm, table_hbm, out_hbm,
             idx_v, pe_v, r0, r1, r2, r3,
             sg0, sg1, sg2, sg3, sw0, sw1, sw2, sw3):
    rows = (r0, r1, r2, r3)
    sem_g = (sg0, sg1, sg2, sg3)
    sem_w = (sw0, sw1, sw2, sw3)
    cid = lax.axis_index("c")
    sid = lax.axis_index("s")
    wid = sid * _NC + cid
    base_seq = wid * _SEQ_PER_W

    pltpu.sync_copy(pe_hbm, pe_v)
    pltpu.sync_copy(idx_hbm.at[pl.ds(base_seq, _SEQ_PER_W)], idx_v)

    def fire_gather(c, b):
        for j in range(_GATHERS):
            pltpu.async_copy(
                table_hbm.at[idx_v.at[c, j]],
                rows[b].at[pl.ds(j * _IDXW, _IDXW)],
                sem_g[b],
            )

    def drain_gather(c, b):
        for j in range(_GATHERS):
            pltpu.make_async_copy(
                table_hbm.at[idx_v.at[c, j]],
                rows[b].at[pl.ds(j * _IDXW, _IDXW)],
                sem_g[b],
            ).wait()

    def out_copy(c, b):
        return pltpu.make_async_copy(
            rows[b], out_hbm.at[pl.ds((base_seq + c) * _S, _S)], sem_w[b])

    for b in range(_PDIST):
        fire_gather(b, b)

    def group_body(g, carry):
        c0 = g * _NBUF
        for b in range(_NBUF):
            c = c0 + b
            drain_gather(c, b)

            @plsc.parallel_loop(0, _S, unroll=4)
            def _(i):
                for k in range(_VECS_PER_ROW):
                    sl = pl.ds(k * 16, 16)
                    rows[b][i, sl] = rows[b][i, sl] * _SCALE + pe_v[i, sl]
            out_copy(c, b).start()

            f = c + _PDIST
            fb = (b + _PDIST) % _NBUF
            if b == 0:
                @pl.when(g >= 1)
                def _():
                    out_copy(f - _NBUF, fb).wait()
                fire_gather(f, fb)
            else:
                @pl.when(g < _NGROUPS - 1)
                def _():
                    out_copy(f - _NBUF, fb).wait()
                    fire_gather(f, fb)
        return carry

    lax.fori_loop(0, _NGROUPS, group_body, 0)

    for b in range(_NBUF):
        out_copy(_SEQ_PER_W - _NBUF + b, b).wait()


@jax.jit
def _music_embedding(idx3d, pe_s, table):
    mesh = plsc.VectorSubcoreMesh(
        core_axis_name="c", subcore_axis_name="s",
        num_cores=_NC, num_subcores=_NS,
    )
    run = pl.kernel(
        _sc_body,
        out_type=jax.ShapeDtypeStruct((_ROWS, _D), jnp.float32),
        mesh=mesh,
        scratch_types=(
            [pltpu.VMEM((_SEQ_PER_W, _GATHERS, _IDXW), jnp.int32),
             pltpu.VMEM((_S, _D), jnp.float32)]
            + [pltpu.VMEM((_S, _D), jnp.float32) for _ in range(_NBUF)]
            + [pltpu.SemaphoreType.DMA for _ in range(2 * _NBUF)]
        ),
        compiler_params=pltpu.CompilerParams(use_tc_tiling_on_sc=False),
    )
    return run(idx3d, pe_s, table)


def kernel(token_ids, table, pe):
    idx3d = token_ids.reshape(_B, _GATHERS, _IDXW).astype(jnp.int32)
    pe_s = pe[:_S]
    out = _music_embedding(idx3d, pe_s, table)
    return out.reshape(_B, _S, _D)


# speedup vs baseline: 4.1453x; 1.0005x over previous
import jax
import jax.numpy as jnp
from jax import lax
from jax.experimental import pallas as pl
from jax.experimental.pallas import tpu as pltpu
from jax.experimental.pallas import tpu_sc as plsc

_VOCAB = 100000
_D = 64
_S = 200
_B = 4096
_NC = 2
_NS = 16
_NW = _NC * _NS
_ROWS = _B * _S
_SEQ_PER_W = _B // _NW
_IDXW = 100
_GATHERS = _S // _IDXW
_NBUF = 4
_PDIST = _NBUF - 1
_NGROUPS = _SEQ_PER_W // _NBUF
_SCALE = 8.0
_VECS_PER_ROW = _D // 16


def _sc_body(idx_hbm, pe_hbm, table_hbm, out_hbm,
             idx_v, pe_v, r0, r1, r2, r3,
             sg0, sg1, sg2, sg3, sw0, sw1, sw2, sw3):
    rows = (r0, r1, r2, r3)
    sem_g = (sg0, sg1, sg2, sg3)
    sem_w = (sw0, sw1, sw2, sw3)
    cid = lax.axis_index("c")
    sid = lax.axis_index("s")
    wid = sid * _NC + cid
    base_seq = wid * _SEQ_PER_W

    pltpu.sync_copy(pe_hbm, pe_v)
    pltpu.sync_copy(idx_hbm.at[pl.ds(base_seq, _SEQ_PER_W)], idx_v)

    def fire_gather(c, b):
        for j in range(_GATHERS):
            pltpu.async_copy(
                table_hbm.at[idx_v.at[c, j]],
                rows[b].at[pl.ds(j * _IDXW, _IDXW)],
                sem_g[b],
            )

    def drain_gather(c, b):
        for j in range(_GATHERS):
            pltpu.make_async_copy(
                table_hbm.at[idx_v.at[c, j]],
                rows[b].at[pl.ds(j * _IDXW, _IDXW)],
                sem_g[b],
            ).wait()

    def out_copy(c, b):
        return pltpu.make_async_copy(rows[b], out_hbm.at[base_seq + c], sem_w[b])

    for b in range(_PDIST):
        fire_gather(b, b)

    def group_body(g, carry):
        c0 = g * _NBUF
        for b in range(_NBUF):
            c = c0 + b
            drain_gather(c, b)

            @plsc.parallel_loop(0, _S, unroll=4)
            def _(i):
                for k in range(_VECS_PER_ROW):
                    sl = pl.ds(k * 16, 16)
                    rows[b][i, sl] = rows[b][i, sl] * _SCALE + pe_v[i, sl]
            out_copy(c, b).start()

            f = c + _PDIST
            fb = (b + _PDIST) % _NBUF
            if b == 0:
                @pl.when(g >= 1)
                def _():
                    out_copy(f - _NBUF, fb).wait()
                fire_gather(f, fb)
            else:
                @pl.when(g < _NGROUPS - 1)
                def _():
                    out_copy(f - _NBUF, fb).wait()
                    fire_gather(f, fb)
        return carry

    lax.fori_loop(0, _NGROUPS, group_body, 0)

    for b in range(_NBUF):
        out_copy(_SEQ_PER_W - _NBUF + b, b).wait()


@jax.jit
def _music_embedding(idx3d, pe_s, table):
    mesh = plsc.VectorSubcoreMesh(
        core_axis_name="c", subcore_axis_name="s",
        num_cores=_NC, num_subcores=_NS,
    )
    run = pl.kernel(
        _sc_body,
        out_type=jax.ShapeDtypeStruct((_B, _S, _D), jnp.float32),
        mesh=mesh,
        scratch_types=(
            [pltpu.VMEM((_SEQ_PER_W, _GATHERS, _IDXW), jnp.int32),
             pltpu.VMEM((_S, _D), jnp.float32)]
            + [pltpu.VMEM((_S, _D), jnp.float32) for _ in range(_NBUF)]
            + [pltpu.SemaphoreType.DMA for _ in range(2 * _NBUF)]
        ),
        compiler_params=pltpu.CompilerParams(use_tc_tiling_on_sc=False),
    )
    return run(idx3d, pe_s, table)


def kernel(token_ids, table, pe):
    idx3d = token_ids.reshape(_B, _GATHERS, _IDXW).astype(jnp.int32)
    pe_s = pe[:_S]
    return _music_embedding(idx3d, pe_s, table)
